# speedup vs baseline: 1.1045x; 1.1045x over previous
import functools

import jax
import jax.numpy as jnp
from jax import lax
from jax.experimental import pallas as pl
from jax.experimental.pallas import tpu as pltpu
from jax.experimental.pallas import tpu_sc as plsc

L = 8
K = 1024
D = 128
LANES = 16

NC = 2
NS = 16
NW = NC * NS

CT = 16


def _build(n_tokens):
    tpw = n_tokens // NW
    n_chunks = tpw // CT
    mesh = plsc.VectorSubcoreMesh(core_axis_name="c", subcore_axis_name="s")

    @functools.partial(
        pl.kernel,
        mesh=mesh,
        out_type=jax.ShapeDtypeStruct((n_tokens, D), jnp.float32),
        scratch_types=[
            pltpu.VMEM((CT * L,), jnp.int32),
            pltpu.VMEM((CT * L, D), jnp.float32),
            pltpu.VMEM((CT, D), jnp.float32),
            pltpu.SemaphoreType.DMA,
        ],
    )
    def lookup(idx_hbm, table_hbm, out_hbm, idx_v, rows_v, out_v, sem):
        wid = lax.axis_index("s") * NC + lax.axis_index("c")
        base = wid * tpw
        lvl_off = (lax.iota(jnp.int32, LANES) & (L - 1)) * K

        def chunk_body(ci, _):
            tok0 = base + ci * CT
            pltpu.sync_copy(idx_hbm.at[pl.ds(tok0 * L, CT * L)], idx_v)

            def fix(i, _):
                idx_v[pl.ds(i * LANES, LANES)] = (
                    idx_v[pl.ds(i * LANES, LANES)] + lvl_off)
                return 0
            lax.fori_loop(0, CT * L // LANES, fix, 0)

            pltpu.async_copy(table_hbm.at[idx_v], rows_v, sem).wait()

            def acc_tok(t, _):
                r0 = t * L
                for d in range(D // LANES):
                    sl = pl.ds(d * LANES, LANES)
                    s = rows_v[r0, sl]
                    for l in range(1, L):
                        s = s + rows_v[r0 + l, sl]
                    out_v[t, sl] = s
                return 0
            lax.fori_loop(0, CT, acc_tok, 0)

            pltpu.sync_copy(out_v, out_hbm.at[pl.ds(tok0, CT)])
            return 0

        lax.fori_loop(0, n_chunks, chunk_body, 0)

    return lookup


def kernel(x_list, weight):
    b, t, l = x_list.shape
    n = b * t
    idx = x_list.reshape(n * l)
    table = weight.reshape(l * K, D)
    out = _build(n)(idx, table)
    return out.reshape(b, t, D)


# speedup vs baseline: 1.6799x; 1.5210x over previous
import functools

import jax
import jax.numpy as jnp
from jax import lax
from jax.experimental import pallas as pl
from jax.experimental.pallas import tpu as pltpu
from jax.experimental.pallas import tpu_sc as plsc

L = 8
K = 1024
D = 128
LANES = 16

NC = 2
NS = 16
NW = NC * NS

CT = 16
NBUF = 2


def _build(n_tokens):
    tpw = n_tokens // NW
    n_chunks = tpw // CT
    mesh = plsc.VectorSubcoreMesh(core_axis_name="c", subcore_axis_name="s")

    @functools.partial(
        pl.kernel,
        mesh=mesh,
        out_type=jax.ShapeDtypeStruct((n_tokens, D), jnp.float32),
        scratch_types=[
            pltpu.VMEM((n_chunks, CT * L), jnp.int32),
            pltpu.VMEM((NBUF, CT * L, D), jnp.float32),
            pltpu.VMEM((CT, D), jnp.float32),
            pltpu.SemaphoreType.DMA,
            pltpu.SemaphoreType.DMA,
        ],
    )
    def lookup(idx_hbm, table_hbm, out_hbm, idx_v, rows_v, out_v, sem0, sem1):
        sems = (sem0, sem1)
        wid = lax.axis_index("s") * NC + lax.axis_index("c")
        base = wid * tpw
        lvl_off = (lax.iota(jnp.int32, LANES) & (L - 1)) * K

        pltpu.sync_copy(idx_hbm.at[wid], idx_v)

        def fix_chunk(ci, _):
            def fix(j, _):
                sl = pl.ds(j * LANES, LANES)
                idx_v[ci, sl] = idx_v[ci, sl] + lvl_off
                return 0
            lax.fori_loop(0, CT * L // LANES, fix, 0)
            return 0
        lax.fori_loop(0, n_chunks, fix_chunk, 0)

        def gather(ci, b):
            pltpu.async_copy(table_hbm.at[idx_v.at[ci]], rows_v.at[b], sems[b])

        def gather_wait(ci, b):
            pltpu.make_async_copy(
                table_hbm.at[idx_v.at[ci]], rows_v.at[b], sems[b]).wait()

        def process(ci, b):
            gather_wait(ci, b)

            def acc_tok(t, _):
                for d in range(D // LANES):
                    sl = pl.ds(d * LANES, LANES)
                    s = rows_v[b, t * L, sl]
                    for l in range(1, L):
                        s = s + rows_v[b, t * L + l, sl]
                    out_v[t, sl] = s
                return 0
            lax.fori_loop(0, CT, acc_tok, 0)

            pltpu.sync_copy(out_v, out_hbm.at[pl.ds(base + ci * CT, CT)])

        gather(0, 0)

        def chunk_pair(g, _):
            ci = g * NBUF
            for b in range(NBUF):

                @pl.when(ci + b + 1 < n_chunks)
                def _():
                    gather(ci + b + 1, (b + 1) % NBUF)

                process(ci + b, b)
            return 0
        lax.fori_loop(0, n_chunks // NBUF, chunk_pair, 0)

    return lookup


def kernel(x_list, weight):
    b, t, l = x_list.shape
    n = b * t
    n_chunks = n // NW // CT
    idx = x_list.reshape(NW, n_chunks, CT * L)
    table = weight.reshape(l * K, D)
    out = _build(n)(idx, table)
    return out.reshape(b, t, D)


# speedup vs baseline: 2.0347x; 1.2112x over previous
import functools

import jax
import jax.numpy as jnp
from jax import lax
from jax.experimental import pallas as pl
from jax.experimental.pallas import tpu as pltpu
from jax.experimental.pallas import tpu_sc as plsc

L = 8
K = 1024
D = 128
LANES = 16

NC = 2
NS = 16
NW = NC * NS

CT = 16
NBUF = 2


def _build(n_tokens):
    tpw = n_tokens // NW
    n_chunks = tpw // CT
    mesh = plsc.VectorSubcoreMesh(core_axis_name="c", subcore_axis_name="s")

    @functools.partial(
        pl.kernel,
        mesh=mesh,
        out_type=jax.ShapeDtypeStruct((n_tokens, D), jnp.float32),
        scratch_types=[
            pltpu.VMEM((n_chunks, CT * L), jnp.int32),
            pltpu.VMEM((NBUF, CT * L, D), jnp.float32),
            pltpu.VMEM((NBUF, CT, D), jnp.float32),
            pltpu.SemaphoreType.DMA,
            pltpu.SemaphoreType.DMA,
            pltpu.SemaphoreType.DMA,
            pltpu.SemaphoreType.DMA,
        ],
    )
    def lookup(idx_hbm, table_hbm, out_hbm, idx_v, rows_v, out_v,
               gsem0, gsem1, osem0, osem1):
        gsems = (gsem0, gsem1)
        osems = (osem0, osem1)
        wid = lax.axis_index("s") * NC + lax.axis_index("c")
        base = wid * tpw
        lvl_off = (lax.iota(jnp.int32, LANES) & (L - 1)) * K

        pltpu.sync_copy(idx_hbm.at[wid], idx_v)

        @plsc.parallel_loop(0, tpw * L // LANES, 1, unroll=4)
        def _fix(i):
            ci = i // (CT * L // LANES)
            sl = pl.ds((i % (CT * L // LANES)) * LANES, LANES)
            idx_v[ci, sl] = idx_v[ci, sl] + lvl_off

        def gather(ci, b):
            pltpu.async_copy(table_hbm.at[idx_v.at[ci]], rows_v.at[b], gsems[b])

        def gather_wait(ci, b):
            pltpu.make_async_copy(
                table_hbm.at[idx_v.at[ci]], rows_v.at[b], gsems[b]).wait()

        def out_start(ci, b):
            pltpu.async_copy(
                out_v.at[b], out_hbm.at[pl.ds(base + ci * CT, CT)], osems[b])

        def out_wait(ci, b):
            pltpu.make_async_copy(
                out_v.at[b], out_hbm.at[pl.ds(base + ci * CT, CT)],
                osems[b]).wait()

        def process(ci, b):
            gather_wait(ci, b)

            @pl.when(ci >= NBUF)
            def _():
                out_wait(ci - NBUF, b)

            @plsc.parallel_loop(0, CT, 1, unroll=2)
            def _acc(t):
                for d in range(D // LANES):
                    sl = pl.ds(d * LANES, LANES)
                    s = rows_v[b, t * L, sl]
                    for l in range(1, L):
                        s = s + rows_v[b, t * L + l, sl]
                    out_v[b, t, sl] = s

            out_start(ci, b)

        gather(0, 0)

        def chunk_pair(g, _):
            ci = g * NBUF
            for b in range(NBUF):

                @pl.when(ci + b + 1 < n_chunks)
                def _():
                    gather(ci + b + 1, (b + 1) % NBUF)

                process(ci + b, b)
            return 0
        lax.fori_loop(0, n_chunks // NBUF, chunk_pair, 0)

        for b in range(NBUF):
            out_wait(n_chunks - NBUF + b, b)

    return lookup


def kernel(x_list, weight):
    b, t, l = x_list.shape
    n = b * t
    n_chunks = n // NW // CT
    idx = x_list.reshape(NW, n_chunks, CT * L)
    table = weight.reshape(l * K, D)
    out = _build(n)(idx, table)
    return out.reshape(b, t, D)
